# baseline (speedup 1.0000x reference)
import jax
import jax.numpy as jnp
from jax.experimental import pallas as pl
from jax.experimental.pallas import tpu as pltpu

E = 8
NC = 80
REG_MAX = 16
C = 192
B = 16
NO = NC + 4 * REG_MAX
HW0, HW1, HW2 = 4096, 1024, 256
HWT = HW0 + HW1 + HW2
H0 = HW0 // 2

SLABS = ((0, 2048), (2048, 4096), (4096, HWT))


def _moe_kernel(idx_ref, x0a_ref, x0b_ref, x1_ref, x2_ref,
                w0_ref, w1_ref, w2_ref, c0_ref, c1_ref, c2_ref,
                out_ref, obuf_ref, sems):
    b = pl.program_id(0)
    s = jax.lax.rem(b, 2)

    def out_copies(slot, row):
        return [
            pltpu.make_async_copy(
                obuf_ref.at[slot, :, lo:hi],
                out_ref.at[row, :, lo:hi],
                sems.at[slot, k],
            )
            for k, (lo, hi) in enumerate(SLABS)
        ]

    @pl.when(b >= 2)
    def _():
        for cp in out_copies(s, b - 2):
            cp.wait()

    def dot16(w_ref, x):
        return jnp.dot(w_ref[0].astype(jnp.bfloat16), x.astype(jnp.bfloat16),
                       preferred_element_type=jnp.float32)

    obuf_ref[s, :, 0:H0] = dot16(w0_ref, x0a_ref[0]) + c0_ref[0]
    obuf_ref[s, :, H0:HW0] = dot16(w0_ref, x0b_ref[0]) + c0_ref[0]
    obuf_ref[s, :, HW0:HW0 + HW1] = dot16(w1_ref, x1_ref[0]) + c1_ref[0]
    obuf_ref[s, :, HW0 + HW1:HWT] = dot16(w2_ref, x2_ref[0]) + c2_ref[0]

    for cp in out_copies(s, b):
        cp.start()

    @pl.when(b == B - 1)
    def _():
        for cp in out_copies(s, b):
            cp.wait()
        for cp in out_copies(1 - s, b - 1):
            cp.wait()


def kernel(x0, x1, x2, module_indices, W2_0, b2_0, W3_0, b3_0,
           W2_1, b2_1, W3_1, b3_1, W2_2, b2_2, W3_2, b3_2):
    xs0 = x0.reshape(B, C, HW0)
    xs1 = x1.reshape(B, C, HW1)
    xs2 = x2.reshape(B, C, HW2)
    Ws = [jnp.concatenate([w2, w3], axis=1)
          for w2, w3 in ((W2_0, W3_0), (W2_1, W3_1), (W2_2, W3_2))]
    bs = [jnp.concatenate([bb2, bb3], axis=1)[:, :, None]
          for bb2, bb3 in ((b2_0, b3_0), (b2_1, b3_1), (b2_2, b3_2))]
    idx = module_indices.astype(jnp.int32)

    grid_spec = pltpu.PrefetchScalarGridSpec(
        num_scalar_prefetch=1,
        grid=(B,),
        in_specs=[
            pl.BlockSpec((1, C, H0), lambda b, i: (b, 0, 0)),
            pl.BlockSpec((1, C, H0), lambda b, i: (b, 0, 1)),
            pl.BlockSpec((1, C, HW1), lambda b, i: (b, 0, 0)),
            pl.BlockSpec((1, C, HW2), lambda b, i: (b, 0, 0)),
            pl.BlockSpec((1, NO, C), lambda b, i: (i[b], 0, 0)),
            pl.BlockSpec((1, NO, C), lambda b, i: (i[b], 0, 0)),
            pl.BlockSpec((1, NO, C), lambda b, i: (i[b], 0, 0)),
            pl.BlockSpec((1, NO, 1), lambda b, i: (i[b], 0, 0)),
            pl.BlockSpec((1, NO, 1), lambda b, i: (i[b], 0, 0)),
            pl.BlockSpec((1, NO, 1), lambda b, i: (i[b], 0, 0)),
        ],
        out_specs=pl.BlockSpec(memory_space=pl.MemorySpace.ANY),
        scratch_shapes=[
            pltpu.VMEM((2, NO, HWT), jnp.float32),
            pltpu.SemaphoreType.DMA((2, len(SLABS))),
        ],
    )

    return pl.pallas_call(
        _moe_kernel,
        grid_spec=grid_spec,
        out_shape=jax.ShapeDtypeStruct((B, NO, HWT), jnp.float32),
        compiler_params=pltpu.CompilerParams(
            dimension_semantics=("arbitrary",),
        ),
    )(idx, xs0, xs0, xs1, xs2, Ws[0], Ws[1], Ws[2], bs[0], bs[1], bs[2])
